# baseline (speedup 1.0000x reference)
import functools

import jax
import jax.numpy as jnp
from jax import lax
from jax.experimental import pallas as pl
from jax.experimental.pallas import tpu as pltpu
from jax.experimental.pallas import tpu_sc as plsc

_NC = 2
_NS = 16
_CHUNK = 512


def kernel(token_seq, table):
    B, S = token_seq.shape
    n = B * S
    d = table.shape[1]
    idx = token_seq.reshape(n)

    nw = _NC * _NS
    per_w = n // nw
    n_chunks = per_w // _CHUNK

    mesh = plsc.VectorSubcoreMesh(core_axis_name="c", subcore_axis_name="s")

    @functools.partial(
        pl.kernel,
        mesh=mesh,
        out_type=jax.ShapeDtypeStruct((n, d), table.dtype),
        compiler_params=pltpu.CompilerParams(use_tc_tiling_on_sc=False),
        scratch_types=[
            pltpu.VMEM((_CHUNK,), jnp.int32),
            pltpu.VMEM((_CHUNK, d), jnp.float32),
            pltpu.SemaphoreType.DMA,
        ],
    )
    def gather_kernel(table_hbm, idx_hbm, out_hbm, idx_v, rows_v, sem):
        wid = lax.axis_index("s") * _NC + lax.axis_index("c")
        base = wid * per_w

        @pl.loop(0, n_chunks)
        def _(c):
            off = base + c * _CHUNK
            pltpu.sync_copy(idx_hbm.at[pl.ds(off, _CHUNK)], idx_v)
            pltpu.async_copy(table_hbm.at[idx_v], rows_v, sem).wait()
            pltpu.sync_copy(rows_v, out_hbm.at[pl.ds(off, _CHUNK)])

    out = gather_kernel(table, idx)
    return out.reshape(B, S, d)


# speedup vs baseline: 1.1237x; 1.1237x over previous
import functools

import jax
import jax.numpy as jnp
from jax import lax
from jax.experimental import pallas as pl
from jax.experimental.pallas import tpu as pltpu
from jax.experimental.pallas import tpu_sc as plsc

_NC = 2
_NS = 16
_CHUNK = 512
_BL = 1024


def _transpose_table(tbl_t):
    v = tbl_t.shape[1]
    g = (v + 2 * _BL - 1) // (2 * _BL)

    def body(a_ref, b_ref, out_ref):
        out_ref[:, 0:64] = a_ref[...].T
        out_ref[:, 64:128] = b_ref[...].T

    return pl.pallas_call(
        body,
        grid=(g,),
        in_specs=[
            pl.BlockSpec((64, _BL), lambda i: (0, 2 * i)),
            pl.BlockSpec(
                (64, _BL), lambda i: (0, lax.min(2 * i + 1, 2 * g - 3))
            ),
        ],
        out_specs=pl.BlockSpec((_BL, 128), lambda i: (i, 0)),
        out_shape=jax.ShapeDtypeStruct((g * _BL, 128), jnp.float32),
        compiler_params=pltpu.CompilerParams(
            dimension_semantics=("parallel",),
        ),
    )(tbl_t, tbl_t)


def kernel(token_seq, table):
    B, S = token_seq.shape
    n = B * S
    d = table.shape[1]

    tbl_t = jnp.swapaxes(table, 0, 1)
    z = _transpose_table(tbl_t)
    tbl_lin = z.reshape(z.shape[0] * 2, d)

    j = token_seq.reshape(n)
    idx = (j & -2048) | ((j & 1023) << 1) | ((j >> 10) & 1)

    nw = _NC * _NS
    per_w = n // nw
    n_chunks = per_w // _CHUNK

    mesh = plsc.VectorSubcoreMesh(core_axis_name="c", subcore_axis_name="s")

    @functools.partial(
        pl.kernel,
        mesh=mesh,
        out_type=jax.ShapeDtypeStruct((n, d), table.dtype),
        compiler_params=pltpu.CompilerParams(use_tc_tiling_on_sc=False),
        scratch_types=[
            pltpu.VMEM((_CHUNK,), jnp.int32),
            pltpu.VMEM((_CHUNK, d), jnp.float32),
            pltpu.SemaphoreType.DMA,
        ],
    )
    def gather_kernel(table_hbm, idx_hbm, out_hbm, idx_v, rows_v, sem):
        wid = lax.axis_index("s") * _NC + lax.axis_index("c")
        base = wid * per_w

        @pl.loop(0, n_chunks)
        def _(c):
            off = base + c * _CHUNK
            pltpu.sync_copy(idx_hbm.at[pl.ds(off, _CHUNK)], idx_v)
            pltpu.async_copy(table_hbm.at[idx_v], rows_v, sem).wait()
            pltpu.sync_copy(rows_v, out_hbm.at[pl.ds(off, _CHUNK)])

    out = gather_kernel(tbl_lin, idx)
    return out.reshape(B, S, d)


# speedup vs baseline: 1.3356x; 1.1886x over previous
import functools

import jax
import jax.numpy as jnp
from jax import lax
from jax.experimental import pallas as pl
from jax.experimental.pallas import tpu as pltpu
from jax.experimental.pallas import tpu_sc as plsc

_NC = 2
_NS = 16
_CHUNK = 512
_BL = 4096


def _transpose_table(tbl_t):
    v = tbl_t.shape[1]
    g = (v + 2 * _BL - 1) // (2 * _BL)
    g2 = (g + 1) // 2
    gtot = 2 * g2

    def tbody(a_vmem, b_vmem, out_vmem):
        out_vmem[:, 0:64] = a_vmem[...].T
        out_vmem[:, 64:128] = b_vmem[...].T

    mesh_tc = pltpu.create_tensorcore_mesh("core", num_cores=2)

    @functools.partial(
        pl.kernel,
        mesh=mesh_tc,
        out_type=jax.ShapeDtypeStruct((gtot * _BL, 128), jnp.float32),
    )
    def tkern(tbl_hbm, z_hbm):
        pltpu.emit_pipeline(
            tbody,
            grid=(gtot,),
            in_specs=[
                pl.BlockSpec(
                    (64, _BL), lambda i: (0, lax.min(2 * i, 2 * g - 2))
                ),
                pl.BlockSpec(
                    (64, _BL), lambda i: (0, lax.min(2 * i + 1, 2 * g - 3))
                ),
            ],
            out_specs=[pl.BlockSpec((_BL, 128), lambda i: (i, 0))],
            core_axis_name="core",
            dimension_semantics=(pltpu.PARALLEL,),
        )(tbl_hbm, tbl_hbm, z_hbm)

    return tkern(tbl_t)


def kernel(token_seq, table):
    B, S = token_seq.shape
    n = B * S
    d = table.shape[1]

    tbl_t = jnp.swapaxes(table, 0, 1)
    z = _transpose_table(tbl_t)
    tbl_lin = z.reshape(z.shape[0] * 2, d)

    j = token_seq.reshape(n)
    lb = _BL.bit_length() - 1
    idx = (j & -(2 * _BL)) | ((j & (_BL - 1)) << 1) | ((j >> lb) & 1)

    nw = _NC * _NS
    per_w = n // nw
    n_chunks = per_w // _CHUNK

    mesh = plsc.VectorSubcoreMesh(core_axis_name="c", subcore_axis_name="s")

    @functools.partial(
        pl.kernel,
        mesh=mesh,
        out_type=jax.ShapeDtypeStruct((n, d), table.dtype),
        compiler_params=pltpu.CompilerParams(use_tc_tiling_on_sc=False),
        scratch_types=[
            pltpu.VMEM((_CHUNK,), jnp.int32),
            pltpu.VMEM((_CHUNK, d), jnp.float32),
            pltpu.SemaphoreType.DMA,
        ],
    )
    def gather_kernel(table_hbm, idx_hbm, out_hbm, idx_v, rows_v, sem):
        wid = lax.axis_index("s") * _NC + lax.axis_index("c")
        base = wid * per_w

        @pl.loop(0, n_chunks)
        def _(c):
            off = base + c * _CHUNK
            pltpu.sync_copy(idx_hbm.at[pl.ds(off, _CHUNK)], idx_v)
            pltpu.async_copy(table_hbm.at[idx_v], rows_v, sem).wait()
            pltpu.sync_copy(rows_v, out_hbm.at[pl.ds(off, _CHUNK)])

    out = gather_kernel(tbl_lin, idx)
    return out.reshape(B, S, d)


# speedup vs baseline: 1.3857x; 1.0375x over previous
import functools

import jax
import jax.numpy as jnp
from jax import lax
from jax.experimental import pallas as pl
from jax.experimental.pallas import tpu as pltpu
from jax.experimental.pallas import tpu_sc as plsc

_NC = 2
_NS = 16
_CHUNK = 512
_BL = 4096


def _transpose_table(tbl_t):
    v = tbl_t.shape[1]
    g = (v + 2 * _BL - 1) // (2 * _BL)
    g2 = (g + 1) // 2
    gtot = 2 * g2

    def tbody(a_vmem, b_vmem, out_vmem):
        out_vmem[:, 0:64] = a_vmem[...].T
        out_vmem[:, 64:128] = b_vmem[...].T

    mesh_tc = pltpu.create_tensorcore_mesh("core", num_cores=2)

    @functools.partial(
        pl.kernel,
        mesh=mesh_tc,
        out_type=jax.ShapeDtypeStruct((gtot * _BL, 128), jnp.float32),
    )
    def tkern(tbl_hbm, z_hbm):
        pltpu.emit_pipeline(
            tbody,
            grid=(gtot,),
            in_specs=[
                pl.BlockSpec(
                    (64, _BL), lambda i: (0, lax.min(2 * i, 2 * g - 2))
                ),
                pl.BlockSpec(
                    (64, _BL), lambda i: (0, lax.min(2 * i + 1, 2 * g - 3))
                ),
            ],
            out_specs=[pl.BlockSpec((_BL, 128), lambda i: (i, 0))],
            core_axis_name="core",
            dimension_semantics=(pltpu.PARALLEL,),
        )(tbl_hbm, tbl_hbm, z_hbm)

    return tkern(tbl_t)


def _transpose_out(pairs, B, S, d):
    gpb = B // 512

    def tbody(in_vmem, out_vmem):
        for kk in range(gpb):
            t = in_vmem[256 * kk : 256 * kk + 256, :].T
            out_vmem[0, :, 512 * kk : 512 * kk + 256] = t[0:64]
            out_vmem[0, :, 512 * kk + 256 : 512 * kk + 512] = t[64:128]

    mesh_tc = pltpu.create_tensorcore_mesh("core", num_cores=2)

    @functools.partial(
        pl.kernel,
        mesh=mesh_tc,
        out_type=jax.ShapeDtypeStruct((S, d, B), jnp.float32),
    )
    def tkern(in_hbm, out_hbm):
        pltpu.emit_pipeline(
            tbody,
            grid=(S,),
            in_specs=[
                pl.BlockSpec((B // 2, 128), lambda s: (s, 0)),
            ],
            out_specs=[pl.BlockSpec((1, d, B), lambda s: (s, 0, 0))],
            core_axis_name="core",
            dimension_semantics=(pltpu.PARALLEL,),
        )(in_hbm, out_hbm)

    return tkern(pairs)


def kernel(token_seq, table):
    B, S = token_seq.shape
    n = B * S
    d = table.shape[1]

    tbl_t = jnp.swapaxes(table, 0, 1)
    z = _transpose_table(tbl_t)
    tbl_lin = z.reshape(z.shape[0] * 2, d)

    t = token_seq.T
    u = t.reshape(S, B // 512, 2, 256)
    j = u.transpose(0, 1, 3, 2).reshape(n)
    lb = _BL.bit_length() - 1
    idx = (j & -(2 * _BL)) | ((j & (_BL - 1)) << 1) | ((j >> lb) & 1)

    nw = _NC * _NS
    per_w = n // nw
    n_chunks = per_w // _CHUNK

    mesh = plsc.VectorSubcoreMesh(core_axis_name="c", subcore_axis_name="s")

    @functools.partial(
        pl.kernel,
        mesh=mesh,
        out_type=jax.ShapeDtypeStruct((n, d), table.dtype),
        compiler_params=pltpu.CompilerParams(use_tc_tiling_on_sc=False),
        scratch_types=[
            pltpu.VMEM((_CHUNK,), jnp.int32),
            pltpu.VMEM((_CHUNK, d), jnp.float32),
            pltpu.SemaphoreType.DMA,
        ],
    )
    def gather_kernel(table_hbm, idx_hbm, out_hbm, idx_v, rows_v, sem):
        wid = lax.axis_index("s") * _NC + lax.axis_index("c")
        base = wid * per_w

        @pl.loop(0, n_chunks)
        def _(c):
            off = base + c * _CHUNK
            pltpu.sync_copy(idx_hbm.at[pl.ds(off, _CHUNK)], idx_v)
            pltpu.async_copy(table_hbm.at[idx_v], rows_v, sem).wait()
            pltpu.sync_copy(rows_v, out_hbm.at[pl.ds(off, _CHUNK)])

    out = gather_kernel(tbl_lin, idx)
    out3 = _transpose_out(out.reshape(n // 2, 2 * d), B, S, d)
    return jnp.transpose(out3, (2, 0, 1))


# speedup vs baseline: 1.8339x; 1.3235x over previous
import functools

import jax
import jax.numpy as jnp
from jax import lax
from jax.experimental import pallas as pl
from jax.experimental.pallas import tpu as pltpu
from jax.experimental.pallas import tpu_sc as plsc

_NC = 2
_NS = 16
_CHUNK = 512
_BL = 4096


def _transpose_table(tbl_t):
    v = tbl_t.shape[1]
    g = (v + 2 * _BL - 1) // (2 * _BL)
    g2 = (g + 1) // 2
    gtot = 2 * g2

    def tbody(a_vmem, b_vmem, out_vmem):
        out_vmem[:, 0:64] = a_vmem[...].T
        out_vmem[:, 64:128] = b_vmem[...].T

    mesh_tc = pltpu.create_tensorcore_mesh("core", num_cores=2)

    @functools.partial(
        pl.kernel,
        mesh=mesh_tc,
        out_type=jax.ShapeDtypeStruct((gtot * _BL, 128), jnp.float32),
    )
    def tkern(tbl_hbm, z_hbm):
        pltpu.emit_pipeline(
            tbody,
            grid=(gtot,),
            in_specs=[
                pl.BlockSpec(
                    (64, _BL), lambda i: (0, lax.min(2 * i, 2 * g - 2))
                ),
                pl.BlockSpec(
                    (64, _BL), lambda i: (0, lax.min(2 * i + 1, 2 * g - 3))
                ),
            ],
            out_specs=[pl.BlockSpec((_BL, 128), lambda i: (i, 0))],
            core_axis_name="core",
            dimension_semantics=(pltpu.PARALLEL,),
        )(tbl_hbm, tbl_hbm, z_hbm)

    return tkern(tbl_t)


def _transpose_out(pairs, B, S, d):
    gpb = B // 512

    def tbody(in_vmem, out_vmem):
        for kk in range(gpb):
            t = in_vmem[256 * kk : 256 * kk + 256, :].T
            out_vmem[0, :, 512 * kk : 512 * kk + 256] = t[0:64]
            out_vmem[0, :, 512 * kk + 256 : 512 * kk + 512] = t[64:128]

    mesh_tc = pltpu.create_tensorcore_mesh("core", num_cores=2)

    @functools.partial(
        pl.kernel,
        mesh=mesh_tc,
        out_type=jax.ShapeDtypeStruct((S, d, B), jnp.float32),
    )
    def tkern(in_hbm, out_hbm):
        pltpu.emit_pipeline(
            tbody,
            grid=(S,),
            in_specs=[
                pl.BlockSpec((B // 2, 128), lambda s: (s, 0)),
            ],
            out_specs=[pl.BlockSpec((1, d, B), lambda s: (s, 0, 0))],
            core_axis_name="core",
            dimension_semantics=(pltpu.PARALLEL,),
        )(in_hbm, out_hbm)

    return tkern(pairs)


def kernel(token_seq, table):
    B, S = token_seq.shape
    n = B * S
    d = table.shape[1]

    tbl_t = jnp.swapaxes(table, 0, 1)
    z = _transpose_table(tbl_t)
    tbl_lin = z.reshape(z.shape[0] * 2, d)

    j = token_seq.T.reshape(n)
    lb = _BL.bit_length() - 1
    idx = (j & -(2 * _BL)) | ((j & (_BL - 1)) << 1) | ((j >> lb) & 1)

    nw = _NC * _NS
    per_w = n // nw
    n_chunks = per_w // _CHUNK

    mesh = plsc.VectorSubcoreMesh(core_axis_name="c", subcore_axis_name="s")

    @functools.partial(
        pl.kernel,
        mesh=mesh,
        out_type=jax.ShapeDtypeStruct((n, d), table.dtype),
        compiler_params=pltpu.CompilerParams(
            use_tc_tiling_on_sc=False, needs_layout_passes=False
        ),
        scratch_types=[
            pltpu.VMEM((_CHUNK,), jnp.int32),
            pltpu.VMEM((_CHUNK,), jnp.int32),
            pltpu.VMEM((_CHUNK, d), jnp.float32),
            pltpu.SemaphoreType.DMA,
        ],
    )
    def gather_kernel(table_hbm, idx_hbm, out_hbm, idx_n, idx_v, rows_v, sem):
        wid = lax.axis_index("s") * _NC + lax.axis_index("c")
        base = wid * per_w
        half = _CHUNK // 2

        @pl.loop(0, n_chunks)
        def _(c):
            off = base + c * _CHUNK
            pltpu.sync_copy(idx_hbm.at[pl.ds(off, _CHUNK)], idx_n)

            @pl.loop(0, half // 16)
            def _(mv):
                pos = (lax.iota(jnp.int32, 16) + mv * 16) * 2
                a = idx_n[pl.ds(mv * 16, 16)]
                b = idx_n[pl.ds(half + mv * 16, 16)]
                plsc.store_scatter(idx_v, [pos], a)
                plsc.store_scatter(idx_v, [pos + 1], b)

            pltpu.async_copy(table_hbm.at[idx_v], rows_v, sem).wait()
            pltpu.sync_copy(rows_v, out_hbm.at[pl.ds(off, _CHUNK)])

    out = gather_kernel(tbl_lin, idx)
    out3 = _transpose_out(out.reshape(n // 2, 2 * d), B, S, d)
    return jnp.transpose(out3, (2, 0, 1))


# speedup vs baseline: 2.1256x; 1.1591x over previous
import functools

import jax
import jax.numpy as jnp
from jax import lax
from jax.experimental import pallas as pl
from jax.experimental.pallas import tpu as pltpu
from jax.experimental.pallas import tpu_sc as plsc

_NC = 2
_NS = 16
_CHUNK = 512
_BL = 8192


def _transpose_table(tbl_t):
    v = tbl_t.shape[1]
    g = (v + 2 * _BL - 1) // (2 * _BL)
    g2 = (g + 1) // 2
    gtot = 2 * g2

    def tbody(a_vmem, b_vmem, out_vmem):
        out_vmem[:, 0:64] = a_vmem[...].T
        out_vmem[:, 64:128] = b_vmem[...].T

    mesh_tc = pltpu.create_tensorcore_mesh("core", num_cores=2)

    @functools.partial(
        pl.kernel,
        mesh=mesh_tc,
        out_type=jax.ShapeDtypeStruct((gtot * _BL, 128), jnp.float32),
    )
    def tkern(tbl_hbm, z_hbm):
        pltpu.emit_pipeline(
            tbody,
            grid=(gtot,),
            in_specs=[
                pl.BlockSpec(
                    (64, _BL), lambda i: (0, lax.min(2 * i, 2 * g - 2))
                ),
                pl.BlockSpec(
                    (64, _BL), lambda i: (0, lax.min(2 * i + 1, 2 * g - 3))
                ),
            ],
            out_specs=[pl.BlockSpec((_BL, 128), lambda i: (i, 0))],
            core_axis_name="core",
            dimension_semantics=(pltpu.PARALLEL,),
        )(tbl_hbm, tbl_hbm, z_hbm)

    return tkern(tbl_t)


def _transpose_out(pairs, B, S, d):
    gpb = B // 512

    def tbody(in_vmem, out_vmem):
        for ss in range(2):
            for kk in range(gpb):
                r0 = ss * (B // 2) + 256 * kk
                t = in_vmem[r0 : r0 + 256, :].T
                out_vmem[ss, :, 512 * kk : 512 * kk + 256] = t[0:64]
                out_vmem[ss, :, 512 * kk + 256 : 512 * kk + 512] = t[64:128]

    mesh_tc = pltpu.create_tensorcore_mesh("core", num_cores=2)

    @functools.partial(
        pl.kernel,
        mesh=mesh_tc,
        out_type=jax.ShapeDtypeStruct((S, d, B), jnp.float32),
    )
    def tkern(in_hbm, out_hbm):
        pltpu.emit_pipeline(
            tbody,
            grid=(S // 2,),
            in_specs=[
                pl.BlockSpec((B, 128), lambda s: (s, 0)),
            ],
            out_specs=[pl.BlockSpec((2, d, B), lambda s: (s, 0, 0))],
            core_axis_name="core",
            dimension_semantics=(pltpu.PARALLEL,),
        )(in_hbm, out_hbm)

    return tkern(pairs)


def kernel(token_seq, table):
    B, S = token_seq.shape
    n = B * S
    d = table.shape[1]

    tbl_t = jnp.swapaxes(table, 0, 1)
    z = _transpose_table(tbl_t)
    tbl_lin = z.reshape(z.shape[0] * 2, d)

    j = token_seq.T.reshape(n)
    lb = _BL.bit_length() - 1
    idx = (j & -(2 * _BL)) | ((j & (_BL - 1)) << 1) | ((j >> lb) & 1)

    nw = _NC * _NS
    per_w = n // nw
    n_chunks = per_w // _CHUNK

    mesh = plsc.VectorSubcoreMesh(core_axis_name="c", subcore_axis_name="s")

    @functools.partial(
        pl.kernel,
        mesh=mesh,
        out_type=jax.ShapeDtypeStruct((n, d), table.dtype),
        compiler_params=pltpu.CompilerParams(
            use_tc_tiling_on_sc=False, needs_layout_passes=False
        ),
        scratch_types=[
            pltpu.VMEM((_CHUNK,), jnp.int32),
            pltpu.VMEM((_CHUNK,), jnp.int32),
            pltpu.VMEM((_CHUNK, d), jnp.float32),
            pltpu.SemaphoreType.DMA,
        ],
    )
    def gather_kernel(table_hbm, idx_hbm, out_hbm, idx_n, idx_v, rows_v, sem):
        wid = lax.axis_index("s") * _NC + lax.axis_index("c")
        base = wid * per_w
        half = _CHUNK // 2

        @pl.loop(0, n_chunks)
        def _(c):
            off = base + c * _CHUNK
            pltpu.sync_copy(idx_hbm.at[pl.ds(off, _CHUNK)], idx_n)

            @pl.loop(0, half // 16)
            def _(mv):
                pos = (lax.iota(jnp.int32, 16) + mv * 16) * 2
                a = idx_n[pl.ds(mv * 16, 16)]
                b = idx_n[pl.ds(half + mv * 16, 16)]
                plsc.store_scatter(idx_v, [pos], a)
                plsc.store_scatter(idx_v, [pos + 1], b)

            pltpu.async_copy(table_hbm.at[idx_v], rows_v, sem).wait()
            pltpu.sync_copy(rows_v, out_hbm.at[pl.ds(off, _CHUNK)])

    out = gather_kernel(tbl_lin, idx)
    out3 = _transpose_out(out.reshape(n // 2, 2 * d), B, S, d)
    return jnp.transpose(out3, (2, 0, 1))


# speedup vs baseline: 2.3879x; 1.1234x over previous
import functools

import jax
import jax.numpy as jnp
from jax import lax
from jax.experimental import pallas as pl
from jax.experimental.pallas import tpu as pltpu
from jax.experimental.pallas import tpu_sc as plsc

_NC = 2
_NS = 16
_CHUNK = 1024
_BL = 16384


def _transpose_table(tbl_t):
    v = tbl_t.shape[1]
    g = (v + 2 * _BL - 1) // (2 * _BL)
    g2 = (g + 1) // 2
    gtot = 2 * g2

    def tbody(a_vmem, b_vmem, out_vmem):
        out_vmem[:, 0:64] = a_vmem[...].T
        out_vmem[:, 64:128] = b_vmem[...].T

    mesh_tc = pltpu.create_tensorcore_mesh("core", num_cores=2)

    nlb = (v + _BL - 1) // _BL - 1

    @functools.partial(
        pl.kernel,
        mesh=mesh_tc,
        out_type=jax.ShapeDtypeStruct((gtot * _BL, 128), jnp.float32),
    )
    def tkern(tbl_hbm, z_hbm):
        pltpu.emit_pipeline(
            tbody,
            grid=(gtot,),
            in_specs=[
                pl.BlockSpec(
                    (64, _BL), lambda i: (0, lax.min(2 * i, nlb))
                ),
                pl.BlockSpec(
                    (64, _BL), lambda i: (0, lax.min(2 * i + 1, nlb))
                ),
            ],
            out_specs=[pl.BlockSpec((_BL, 128), lambda i: (i, 0))],
            core_axis_name="core",
            dimension_semantics=(pltpu.PARALLEL,),
        )(tbl_hbm, tbl_hbm, z_hbm)

    return tkern(tbl_t)


def _transpose_out(pairs, B, S, d):
    gpb = B // 512

    def tbody(in_vmem, out_vmem):
        for ss in range(8):
            for kk in range(gpb):
                r0 = ss * (B // 2) + 256 * kk
                t = in_vmem[r0 : r0 + 256, :].T
                out_vmem[ss, :, 512 * kk : 512 * kk + 256] = t[0:64]
                out_vmem[ss, :, 512 * kk + 256 : 512 * kk + 512] = t[64:128]

    mesh_tc = pltpu.create_tensorcore_mesh("core", num_cores=2)

    @functools.partial(
        pl.kernel,
        mesh=mesh_tc,
        out_type=jax.ShapeDtypeStruct((S, d, B), jnp.float32),
    )
    def tkern(in_hbm, out_hbm):
        pltpu.emit_pipeline(
            tbody,
            grid=(S // 8,),
            in_specs=[
                pl.BlockSpec((4 * B, 128), lambda s: (s, 0)),
            ],
            out_specs=[pl.BlockSpec((8, d, B), lambda s: (s, 0, 0))],
            core_axis_name="core",
            dimension_semantics=(pltpu.PARALLEL,),
        )(in_hbm, out_hbm)

    return tkern(pairs)


def kernel(token_seq, table):
    B, S = token_seq.shape
    n = B * S
    d = table.shape[1]

    tbl_t = jnp.swapaxes(table, 0, 1)
    z = _transpose_table(tbl_t)
    tbl_lin = z.reshape(z.shape[0] * 2, d)

    j = token_seq.T.reshape(n)
    lb = _BL.bit_length() - 1
    idx = (j & -(2 * _BL)) | ((j & (_BL - 1)) << 1) | ((j >> lb) & 1)

    nw = _NC * _NS
    per_w = n // nw
    n_chunks = per_w // _CHUNK

    mesh = plsc.VectorSubcoreMesh(core_axis_name="c", subcore_axis_name="s")

    @functools.partial(
        pl.kernel,
        mesh=mesh,
        out_type=jax.ShapeDtypeStruct((n, d), table.dtype),
        compiler_params=pltpu.CompilerParams(
            use_tc_tiling_on_sc=False, needs_layout_passes=False
        ),
        scratch_types=[
            pltpu.VMEM((_CHUNK,), jnp.int32),
            pltpu.VMEM((_CHUNK,), jnp.int32),
            pltpu.VMEM((_CHUNK, d), jnp.float32),
            pltpu.SemaphoreType.DMA,
        ],
    )
    def gather_kernel(table_hbm, idx_hbm, out_hbm, idx_n, idx_v, rows_v, sem):
        wid = lax.axis_index("s") * _NC + lax.axis_index("c")
        base = wid * per_w

        @pl.loop(0, n_chunks)
        def _(c):
            off = base + c * _CHUNK
            pltpu.sync_copy(idx_hbm.at[pl.ds(off, _CHUNK)], idx_n)

            @pl.loop(0, _CHUNK // 32)
            def _(mv):
                grp = mv // 16
                mloc = (mv % 16) * 16
                gbase = grp * 512
                pos = gbase + (lax.iota(jnp.int32, 16) + mloc) * 2
                a = idx_n[pl.ds(gbase + mloc, 16)]
                b = idx_n[pl.ds(gbase + 256 + mloc, 16)]
                plsc.store_scatter(idx_v, [pos], a)
                plsc.store_scatter(idx_v, [pos + 1], b)

            pltpu.async_copy(table_hbm.at[idx_v], rows_v, sem).wait()
            pltpu.sync_copy(rows_v, out_hbm.at[pl.ds(off, _CHUNK)])

    out = gather_kernel(tbl_lin, idx)
    out3 = _transpose_out(out.reshape(n // 2, 2 * d), B, S, d)
    return jnp.transpose(out3, (2, 0, 1))
